# baseline (speedup 1.0000x reference)
import functools

import jax
import jax.numpy as jnp
from jax import lax
from jax.experimental import pallas as pl
from jax.experimental.pallas import tpu as pltpu
from jax.experimental.pallas import tpu_sc as plsc

B = 8
C = 21
HW = 512 * 512
N_IN = B * HW
N_OUT = B * C * HW

NC = 2
NS = 16
NW = NC * NS
PER_W = N_IN // NW
SUB = 8192
NSUB = PER_W // SUB


def _body(x_hbm, out_hbm, in_v, obuf, sem):
    cid = lax.axis_index("c")
    sid = lax.axis_index("s")
    wid = sid * NC + cid
    b = wid // 4
    q = wid % 4
    hw_off = q * PER_W

    pltpu.sync_copy(x_hbm.at[pl.ds(wid * PER_W, PER_W)], in_v)

    for c in range(C):
        out_base = (b * C + c) * HW + hw_off
        for j in range(NSUB):

            def inner(i, _, j=j, c=c):
                v = in_v[pl.ds(j * SUB + i * 16, 16)]
                obuf[0, pl.ds(i * 16, 16)] = jnp.where(
                    v == c, jnp.int32(1), jnp.int32(0)
                )
                return _

            lax.fori_loop(0, SUB // 16, inner, 0)
            pltpu.sync_copy(
                obuf.at[0], out_hbm.at[pl.ds(out_base + j * SUB, SUB)]
            )


@functools.partial(
    pl.kernel,
    out_type=jax.ShapeDtypeStruct((N_OUT,), jnp.int32),
    mesh=plsc.VectorSubcoreMesh(core_axis_name="c", subcore_axis_name="s"),
    scratch_types=[
        pltpu.VMEM((PER_W,), jnp.int32),
        pltpu.VMEM((2, SUB), jnp.int32),
        pltpu.SemaphoreType.DMA,
    ],
)
def _one_hot_sc(x_hbm, out_hbm, in_v, obuf, sem):
    _body(x_hbm, out_hbm, in_v, obuf, sem)


def kernel(x):
    x_flat = x.astype(jnp.int32).reshape(N_IN)
    out_flat = _one_hot_sc(x_flat)
    return out_flat.reshape(B, C, 512, 512)


# speedup vs baseline: 2.4882x; 2.4882x over previous
import functools

import jax
import jax.numpy as jnp
from jax import lax
from jax.experimental import pallas as pl
from jax.experimental.pallas import tpu as pltpu
from jax.experimental.pallas import tpu_sc as plsc

B = 8
C = 21
HW = 512 * 512
N_IN = B * HW
N_OUT = B * C * HW

NC = 2
NS = 16
NW = NC * NS
PER_W = N_IN // NW
SUB = 16384
NSUB = PER_W // SUB
UNROLL = 8


def _body(x_hbm, out_hbm, in_v, buf0, buf1, sem0, sem1):
    cid = lax.axis_index("c")
    sid = lax.axis_index("s")
    wid = sid * NC + cid
    b = wid // 4
    q = wid % 4
    hw_off = q * PER_W

    pltpu.sync_copy(x_hbm.at[pl.ds(wid * PER_W, PER_W)], in_v)

    bufs = (buf0, buf1)
    sems = (sem0, sem1)
    pending = [None, None]

    r = 0
    for c in range(C):
        out_base = (b * C + c) * HW + hw_off
        for j in range(NSUB):
            p = r % 2
            buf, sem = bufs[p], sems[p]
            if pending[p] is not None:
                pending[p].wait()

            joff = j * SUB

            def inner(i, carry, joff=joff, c=c, buf=buf):
                base = i * (16 * UNROLL)
                for u in range(UNROLL):
                    off = base + u * 16
                    v = in_v[pl.ds(joff + off, 16)]
                    buf[pl.ds(off, 16)] = jnp.where(
                        v == c, jnp.int32(1), jnp.int32(0)
                    )
                return carry

            lax.fori_loop(0, SUB // (16 * UNROLL), inner, 0)

            pending[p] = pltpu.async_copy(
                buf, out_hbm.at[pl.ds(out_base + joff, SUB)], sem
            )
            r += 1

    for p in range(2):
        if pending[p] is not None:
            pending[p].wait()


@functools.partial(
    pl.kernel,
    out_type=jax.ShapeDtypeStruct((N_OUT,), jnp.int32),
    mesh=plsc.VectorSubcoreMesh(core_axis_name="c", subcore_axis_name="s"),
    scratch_types=[
        pltpu.VMEM((PER_W,), jnp.int32),
        pltpu.VMEM((SUB,), jnp.int32),
        pltpu.VMEM((SUB,), jnp.int32),
        pltpu.SemaphoreType.DMA,
        pltpu.SemaphoreType.DMA,
    ],
)
def _one_hot_sc(x_hbm, out_hbm, in_v, buf0, buf1, sem0, sem1):
    _body(x_hbm, out_hbm, in_v, buf0, buf1, sem0, sem1)


def kernel(x):
    x_flat = x.astype(jnp.int32).reshape(N_IN)
    out_flat = _one_hot_sc(x_flat)
    return out_flat.reshape(B, C, 512, 512)


# speedup vs baseline: 8.0943x; 3.2531x over previous
import functools

import jax
import jax.numpy as jnp
from jax import lax
from jax.experimental import pallas as pl
from jax.experimental.pallas import tpu as pltpu
from jax.experimental.pallas import tpu_sc as plsc

B = 8
C = 21
H = 512
W = 512

NC = 2
NS = 16
NW = NC * NS
BAND = H // 4
SUBROWS = 32
NSUB = BAND // SUBROWS
VPR = W // 16
UNROLL = 8


def _body(x_hbm, out_hbm, in_v, buf0, buf1, sem0, sem1):
    cid = lax.axis_index("c")
    sid = lax.axis_index("s")
    wid = sid * NC + cid
    b = wid // 4
    q = wid % 4
    row0 = q * BAND

    pltpu.sync_copy(x_hbm.at[b, 0, pl.ds(row0, BAND), :], in_v)

    bufs = (buf0, buf1)
    sems = (sem0, sem1)
    pending = [None, None]

    r = 0
    for c in range(C):
        for j in range(NSUB):
            p = r % 2
            buf, sem = bufs[p], sems[p]
            if pending[p] is not None:
                pending[p].wait()

            def inner(i, carry, j=j, c=c, buf=buf):
                rr = i // (VPR // UNROLL)
                cb = (i % (VPR // UNROLL)) * (16 * UNROLL)
                src_row = j * SUBROWS + rr
                for u in range(UNROLL):
                    v = in_v[src_row, pl.ds(cb + u * 16, 16)]
                    buf[rr, pl.ds(cb + u * 16, 16)] = jnp.where(
                        v == c, jnp.int32(1), jnp.int32(0)
                    )
                return carry

            lax.fori_loop(0, SUBROWS * (VPR // UNROLL), inner, 0)

            pending[p] = pltpu.async_copy(
                buf,
                out_hbm.at[b, c, pl.ds(row0 + j * SUBROWS, SUBROWS), :],
                sem,
            )
            r += 1

    for p in range(2):
        if pending[p] is not None:
            pending[p].wait()


@functools.partial(
    pl.kernel,
    out_type=jax.ShapeDtypeStruct((B, C, H, W), jnp.int32),
    mesh=plsc.VectorSubcoreMesh(core_axis_name="c", subcore_axis_name="s"),
    scratch_types=[
        pltpu.VMEM((BAND, W), jnp.int32),
        pltpu.VMEM((SUBROWS, W), jnp.int32),
        pltpu.VMEM((SUBROWS, W), jnp.int32),
        pltpu.SemaphoreType.DMA,
        pltpu.SemaphoreType.DMA,
    ],
)
def _one_hot_sc(x_hbm, out_hbm, in_v, buf0, buf1, sem0, sem1):
    _body(x_hbm, out_hbm, in_v, buf0, buf1, sem0, sem1)


def kernel(x):
    return _one_hot_sc(x.astype(jnp.int32))
